# baseline (speedup 1.0000x reference)
import math

import jax
import jax.numpy as jnp
from jax import lax
from jax.experimental import pallas as pl
from jax.experimental.pallas import tpu as pltpu
from jax.experimental.pallas import tpu_sc as plsc

D_LANES = 16
NUM_CORES = 2
NUM_SUBCORES = 16
NW = NUM_CORES * NUM_SUBCORES
SBLK = 128


def _make_gather(W: int, S: int, V: int, D: int, ch: int, nbuf: int,
                 unroll: int):
    assert S % (NW * SBLK) == 0
    sreps = S // (NW * SBLK)
    assert SBLK % ch == 0
    per_j = SBLK // ch
    nchunk = W * sreps * per_j
    assert nchunk >= nbuf >= 2
    assert ch % 8 == 0
    assert D % D_LANES == 0
    scale = math.sqrt(float(D))
    vregs_per_row = D // D_LANES
    depth = nbuf - 1
    B = W * S

    mesh = plsc.VectorSubcoreMesh(core_axis_name="c", subcore_axis_name="s")

    @pl.kernel(
        out_type=jax.ShapeDtypeStruct((B, D), jnp.float32),
        mesh=mesh,
        compiler_params=pltpu.CompilerParams(use_tc_tiling_on_sc=True),
        scratch_types=[
            pltpu.VMEM((W, SBLK), jnp.int32),
        ]
        + [pltpu.VMEM((ch, D), jnp.float32) for _ in range(nbuf)]
        + [pltpu.SemaphoreType.DMA for _ in range(2 * nbuf)],
    )
    def gather_scaled(tokens_t_hbm, table_hbm, out_hbm, idx_v, *rest):
        bufs = rest[:nbuf]
        sgs = rest[nbuf:2 * nbuf]
        sos = rest[2 * nbuf:]
        wid = lax.axis_index("s") * NUM_CORES + lax.axis_index("c")
        col0 = wid * (sreps * SBLK)
        pltpu.sync_copy(tokens_t_hbm.at[:, pl.ds(col0, sreps * SBLK)], idx_v)

        def out_row(g):
            jr, h = divmod(g, per_j)
            j, r = divmod(jr, sreps)
            return j * S + col0 + r * SBLK + h * ch

        def gather_start(g):
            b = g % nbuf
            jr, h = divmod(g, per_j)
            return pltpu.async_copy(
                table_hbm.at[idx_v.at[jr, pl.ds(h * ch, ch)]], bufs[b], sgs[b]
            )

        gh = [None] * nchunk
        oh = [None] * nchunk
        for k in range(depth):
            gh[k] = gather_start(k)
        for g in range(nchunk):
            b = g % nbuf
            if g + depth < nchunk:
                if g + depth >= nbuf:
                    oh[g + depth - nbuf].wait()
                gh[g + depth] = gather_start(g + depth)
            gh[g].wait()

            buf = bufs[b]

            @plsc.parallel_loop(0, ch, 1, unroll=unroll)
            def _(r):
                for d in range(vregs_per_row):
                    sl = pl.ds(d * D_LANES, D_LANES)
                    buf[r, sl] = buf[r, sl] * scale

            oh[g] = pltpu.async_copy(
                buf, out_hbm.at[pl.ds(out_row(g), ch)], sos[b]
            )
        for g in range(max(0, nchunk - nbuf), nchunk):
            oh[g].wait()

    return gather_scaled


def kernel(tokens, table):
    assert tokens.ndim == 2
    V, D = table.shape
    S, W = tokens.shape
    tok_t = tokens.T.astype(jnp.int32)
    gather = _make_gather(W, S, V, D, ch=64, nbuf=12, unroll=2)
    out = gather(tok_t, table)
    return out.reshape(W, S, D).transpose(1, 0, 2)


# speedup vs baseline: 1.0543x; 1.0543x over previous
import math

import jax
import jax.numpy as jnp
from jax import lax
from jax.experimental import pallas as pl
from jax.experimental.pallas import tpu as pltpu
from jax.experimental.pallas import tpu_sc as plsc

D_LANES = 16
NUM_CORES = 2
NUM_SUBCORES = 16
NW = NUM_CORES * NUM_SUBCORES
SBLK = 128


def _make_gather(W: int, S: int, V: int, D: int, ch: int, nbuf: int,
                 unroll: int):
    assert S % (NW * SBLK) == 0
    sreps = S // (NW * SBLK)
    assert SBLK % ch == 0
    per_j = SBLK // ch
    nchunk = W * sreps * per_j
    assert nchunk >= nbuf >= 2
    assert ch % 8 == 0
    assert D % D_LANES == 0
    scale = math.sqrt(float(D))
    vregs_per_row = D // D_LANES
    depth = nbuf - 1
    B = W * S

    mesh = plsc.VectorSubcoreMesh(core_axis_name="c", subcore_axis_name="s")

    @pl.kernel(
        out_type=jax.ShapeDtypeStruct((B, D), jnp.float32),
        mesh=mesh,
        compiler_params=pltpu.CompilerParams(use_tc_tiling_on_sc=True),
        scratch_types=[
            pltpu.VMEM((W, SBLK), jnp.int32),
        ]
        + [pltpu.VMEM((ch, D), jnp.float32) for _ in range(nbuf)]
        + [pltpu.SemaphoreType.DMA for _ in range(2 * nbuf)],
    )
    def gather_scaled(tokens_t_hbm, table_hbm, out_hbm, idx_v, *rest):
        bufs = rest[:nbuf]
        sgs = rest[nbuf:2 * nbuf]
        sos = rest[2 * nbuf:]
        wid = lax.axis_index("s") * NUM_CORES + lax.axis_index("c")
        col0 = wid * (sreps * SBLK)
        pltpu.sync_copy(tokens_t_hbm.at[:, pl.ds(col0, sreps * SBLK)], idx_v)

        def out_row(g):
            jr, h = divmod(g, per_j)
            j, r = divmod(jr, sreps)
            return j * S + col0 + r * SBLK + h * ch

        def gather_start(g):
            b = g % nbuf
            jr, h = divmod(g, per_j)
            return pltpu.async_copy(
                table_hbm.at[idx_v.at[jr, pl.ds(h * ch, ch)]], bufs[b], sgs[b]
            )

        gh = [None] * nchunk
        oh = [None] * nchunk
        for k in range(depth):
            gh[k] = gather_start(k)
        for g in range(nchunk):
            b = g % nbuf
            if g + depth < nchunk:
                if g + depth >= nbuf:
                    oh[g + depth - nbuf].wait()
                gh[g + depth] = gather_start(g + depth)
            gh[g].wait()

            buf = bufs[b]

            @plsc.parallel_loop(0, ch, 1, unroll=unroll)
            def _(r):
                for d in range(vregs_per_row):
                    sl = pl.ds(d * D_LANES, D_LANES)
                    buf[r, sl] = buf[r, sl] * scale

            oh[g] = pltpu.async_copy(
                buf, out_hbm.at[pl.ds(out_row(g), ch)], sos[b]
            )
        for g in range(max(0, nchunk - nbuf), nchunk):
            oh[g].wait()

    return gather_scaled


def kernel(tokens, table):
    assert tokens.ndim == 2
    V, D = table.shape
    S, W = tokens.shape
    tok_t = tokens.T.astype(jnp.int32)
    gather = _make_gather(W, S, V, D, ch=128, nbuf=7, unroll=2)
    out = gather(tok_t, table)
    return out.reshape(W, S, D).transpose(1, 0, 2)
